# baseline (speedup 1.0000x reference)
import functools

import jax
import jax.numpy as jnp
from jax import lax
from jax.experimental import pallas as pl
from jax.experimental.pallas import tpu as pltpu
from jax.experimental.pallas import tpu_sc as plsc

N_VISITS = 6000
N_CCSS = 2000
N_ICDS = 2000
N_NODES = N_VISITS + N_CCSS + N_ICDS
N_EDGES = 320000
DIM = 128

_NC, _NS, _L = 2, 16, 16
_NW = _NC * _NS
SROWS = 6016
SENT = 6000
CH = 80
EPW = N_EDGES // _NW
_ROWS_PER_TILE = SROWS // _NS

EV_LCAP = 240
VV_LCAP = 360
OFF_LCAP = 800
EV_T = _L * EV_LCAP
VV_T = _L * VV_LCAP
OFF_T = _L * OFF_LCAP
EV_NCH = EV_T // CH
VV_NCH = VV_T // CH
OFF_NCH = OFF_T // CH
EV_BW = SROWS // _NW
TRASH2 = EV_BW
ACC2_R = EV_BW + 20
BIN_W = 320
TRASH = BIN_W
ACC_R = BIN_W + 16
N_MAX_TILES = (N_VISITS + N_CCSS) // BIN_W
CH2 = 2000
N_CH2 = N_EDGES // CH2

_mesh = plsc.VectorSubcoreMesh(core_axis_name="c", subcore_axis_name="s")


@functools.partial(
    pl.kernel,
    out_type=(
        jax.ShapeDtypeStruct((_NW * EV_T,), jnp.int32),
        jax.ShapeDtypeStruct((_NW * EV_T,), jnp.int32),
        jax.ShapeDtypeStruct((_NW * VV_T,), jnp.int32),
        jax.ShapeDtypeStruct((_NW * VV_T,), jnp.int32),
        jax.ShapeDtypeStruct((_NW * OFF_T,), jnp.int32),
        jax.ShapeDtypeStruct((_NW * OFF_T,), jnp.int32),
    ),
    mesh=_mesh,
    compiler_params=pltpu.CompilerParams(needs_layout_passes=False),
    scratch_types=[
        pltpu.VMEM((CH2,), jnp.int32),
        pltpu.VMEM((CH2,), jnp.int32),
        pltpu.VMEM((CH2,), jnp.int32),
        pltpu.VMEM((CH2,), jnp.int32),
        pltpu.VMEM((EV_T + _L,), jnp.int32),
        pltpu.VMEM((EV_T + _L,), jnp.int32),
        pltpu.VMEM((VV_T + _L,), jnp.int32),
        pltpu.VMEM((VV_T + _L,), jnp.int32),
        pltpu.VMEM((OFF_T + _L,), jnp.int32),
        pltpu.VMEM((OFF_T + _L,), jnp.int32),
        pltpu.VMEM((4 * _L,), jnp.int32),
        pltpu.SemaphoreType.DMA,
        pltpu.SemaphoreType.DMA,
    ],
)
def _classify(h_hbm, t_hbm, evh_hbm, evt_hbm, vvh_hbm, vvt_hbm, offh_hbm, offt_hbm,
              hv0, tv0, hv1, tv1, evh, evt, vvh, vvt, offh, offt, st, sem0, sem1):
    c = lax.axis_index("c")
    s = lax.axis_index("s")
    wid = s * _NC + c
    bin_lo = wid * BIN_W
    bin_hi = bin_lo + BIN_W
    ev_lo = wid * EV_BW
    ev_hi = ev_lo + EV_BW
    lanes = lax.iota(jnp.int32, _L)

    st[pl.ds(_L, _L)] = jnp.zeros((_L,), jnp.int32)
    st[pl.ds(2 * _L, _L)] = jnp.zeros((_L,), jnp.int32)
    st[pl.ds(3 * _L, _L)] = jnp.zeros((_L,), jnp.int32)

    def bin_groups(hv, tv):
        def group(g, carry):
            sl = pl.ds(pl.multiple_of(g * _L, _L), _L)
            hh = hv[sl]
            tt = tv[sl]

            def compact(slot, mask, ref_h, ref_t, lcap, dump, hval):
                cnt = st[pl.ds(slot * _L, _L)]
                ok = mask & (cnt < lcap)
                pos = jnp.where(ok, lanes * lcap + cnt, dump)
                plsc.store_scatter(ref_h, [pos], hval)
                plsc.store_scatter(ref_t, [pos], tt)
                st[pl.ds(slot * _L, _L)] = cnt + ok.astype(jnp.int32)

            m_own = (hh >= bin_lo) & (hh < bin_hi)
            compact(3, m_own, offh, offt, OFF_LCAP, OFF_T, hh - bin_lo)
            inr = (hh >= ev_lo) & (hh < ev_hi)
            hloc = hh - ev_lo
            compact(1, inr & (tt >= N_VISITS), evh, evt, EV_LCAP, EV_T, hloc)
            compact(2, inr & (tt < N_VISITS), vvh, vvt, VV_LCAP, VV_T, hloc)
            return carry

        lax.fori_loop(0, CH2 // _L, group, 0)

    pltpu.async_copy(h_hbm.at[pl.ds(0, CH2)], hv0, sem0)
    pltpu.async_copy(t_hbm.at[pl.ds(0, CH2)], tv0, sem0)

    def pair(i2, carry):
        i = i2 * 2
        pltpu.async_copy(h_hbm.at[pl.ds((i + 1) * CH2, CH2)], hv1, sem1)
        pltpu.async_copy(t_hbm.at[pl.ds((i + 1) * CH2, CH2)], tv1, sem1)
        pltpu.make_async_copy(h_hbm.at[pl.ds(i * CH2, CH2)], hv0, sem0).wait()
        pltpu.make_async_copy(t_hbm.at[pl.ds(i * CH2, CH2)], tv0, sem0).wait()
        bin_groups(hv0, tv0)

        @pl.when(i2 < N_CH2 // 2 - 1)
        def _():
            pltpu.async_copy(h_hbm.at[pl.ds((i + 2) * CH2, CH2)], hv0, sem0)
            pltpu.async_copy(t_hbm.at[pl.ds((i + 2) * CH2, CH2)], tv0, sem0)

        pltpu.make_async_copy(h_hbm.at[pl.ds((i + 1) * CH2, CH2)], hv1, sem1).wait()
        pltpu.make_async_copy(t_hbm.at[pl.ds((i + 1) * CH2, CH2)], tv1, sem1).wait()
        bin_groups(hv1, tv1)
        return carry

    lax.fori_loop(0, N_CH2 // 2, pair, 0)

    def fill(slot, ref_h, ref_t, lcap, dump, pad_h):
        cnt = st[pl.ds(slot * _L, _L)]
        st[pl.ds(0, _L)] = jnp.zeros((_L,), jnp.int32)

        def fbody(i, carry):
            iv = st[pl.ds(0, _L)]
            st[pl.ds(0, _L)] = iv + 1
            need = iv >= cnt
            pos = jnp.where(need, lanes * lcap + iv, dump)
            plsc.store_scatter(ref_h, [pos], pad_h)
            plsc.store_scatter(ref_t, [pos], (lanes * 251 + iv) & 4095)
            return carry

        lax.fori_loop(0, lcap, fbody, 0)

    fill(1, evh, evt, EV_LCAP, EV_T, jnp.zeros((_L,), jnp.int32) + TRASH2)
    fill(2, vvh, vvt, VV_LCAP, VV_T, jnp.zeros((_L,), jnp.int32) + TRASH2)
    fill(3, offh, offt, OFF_LCAP, OFF_T, jnp.zeros((_L,), jnp.int32) + TRASH)

    pltpu.sync_copy(evh.at[pl.ds(0, EV_T)], evh_hbm.at[pl.ds(wid * EV_T, EV_T)])
    pltpu.sync_copy(evt.at[pl.ds(0, EV_T)], evt_hbm.at[pl.ds(wid * EV_T, EV_T)])
    pltpu.sync_copy(vvh.at[pl.ds(0, VV_T)], vvh_hbm.at[pl.ds(wid * VV_T, VV_T)])
    pltpu.sync_copy(vvt.at[pl.ds(0, VV_T)], vvt_hbm.at[pl.ds(wid * VV_T, VV_T)])
    pltpu.sync_copy(offh.at[pl.ds(0, OFF_T)], offh_hbm.at[pl.ds(wid * OFF_T, OFF_T)])
    pltpu.sync_copy(offt.at[pl.ds(0, OFF_T)], offt_hbm.at[pl.ds(wid * OFF_T, OFF_T)])


def _make_seg_sum(n_chunks):
    @functools.partial(
        pl.kernel,
        out_type=jax.ShapeDtypeStruct((SROWS * 256,), jnp.float32),
        mesh=_mesh,
        compiler_params=pltpu.CompilerParams(needs_layout_passes=False),
        scratch_types=[
            pltpu.VMEM((n_chunks, CH), jnp.int32),
            pltpu.VMEM((n_chunks, CH), jnp.int32),
            pltpu.VMEM((CH, 256), jnp.float32),
            pltpu.VMEM((CH, 256), jnp.float32),
            pltpu.VMEM((ACC2_R * 256,), jnp.float32),
            pltpu.SemaphoreType.DMA,
            pltpu.SemaphoreType.DMA,
        ],
    )
    def seg_sum(lh_hbm, lt_hbm, tbl_hbm, out_hbm, hl2d, tl2d, rows0, rows1, acc, sem0, sem1):
        c = lax.axis_index("c")
        s = lax.axis_index("s")
        wid = s * _NC + c
        lanes = lax.iota(jnp.int32, _L)
        zsplat = jnp.zeros((_L,), jnp.float32)

        pltpu.sync_copy(lh_hbm.at[pl.ds(wid * n_chunks, n_chunks)], hl2d)
        pltpu.sync_copy(lt_hbm.at[pl.ds(wid * n_chunks, n_chunks)], tl2d)

        def zbody(r, carry):
            acc[pl.ds(r * _L, _L)] = zsplat
            return carry

        lax.fori_loop(0, ACC2_R * 256 // _L, zbody, 0)

        pltpu.async_copy(tbl_hbm.at[tl2d.at[0]], rows0, sem0)

        def rmw_chunk(i, rows):
            isplat = jnp.zeros((_L,), jnp.int32) + i

            def rmw(j, carry2):
                jsplat = jnp.zeros((_L,), jnp.int32) + j
                hl = plsc.load_gather(hl2d, [isplat, jsplat])[0]
                abase = hl * 256
                for k in range(256 // _L):
                    v = plsc.load_gather(rows, [jsplat, lanes + k * _L])
                    asl = pl.ds(abase + k * _L, _L)
                    acc[asl] = acc[asl] + v
                return carry2

            lax.fori_loop(0, CH, rmw, 0)

        def pair(i2, carry):
            i = i2 * 2
            pltpu.async_copy(tbl_hbm.at[tl2d.at[i + 1]], rows1, sem1)
            pltpu.make_async_copy(tbl_hbm.at[tl2d.at[i]], rows0, sem0).wait()
            rmw_chunk(i, rows0)

            @pl.when(i2 < n_chunks // 2 - 1)
            def _():
                pltpu.async_copy(tbl_hbm.at[tl2d.at[i + 2]], rows0, sem0)

            pltpu.make_async_copy(tbl_hbm.at[tl2d.at[i + 1]], rows1, sem1).wait()
            rmw_chunk(i + 1, rows1)
            return carry

        lax.fori_loop(0, n_chunks // 2, pair, 0)
        pltpu.sync_copy(
            acc.at[pl.ds(0, EV_BW * 256)],
            out_hbm.at[pl.ds(wid * EV_BW * 256, EV_BW * 256)],
        )

    return seg_sum


_seg_sum_ev = _make_seg_sum(EV_NCH)
_seg_sum_vv = _make_seg_sum(VV_NCH)


@functools.partial(
    pl.kernel,
    out_type=jax.ShapeDtypeStruct((_NW * ACC_R * DIM,), jnp.float32),
    mesh=_mesh,
    compiler_params=pltpu.CompilerParams(needs_layout_passes=False),
    scratch_types=[
        pltpu.VMEM((OFF_NCH, CH), jnp.int32),
        pltpu.VMEM((OFF_NCH, CH), jnp.int32),
        pltpu.VMEM((CH, DIM), jnp.float32),
        pltpu.VMEM((CH, DIM), jnp.float32),
        pltpu.VMEM((ACC_R * DIM,), jnp.float32),
        pltpu.SemaphoreType.DMA,
        pltpu.SemaphoreType.DMA,
    ],
)
def _offsets(offh_hbm, offt_hbm, off_hbm, out_hbm, oh2d, ot2d, rows0, rows1, acc, sem0, sem1):
    c = lax.axis_index("c")
    s = lax.axis_index("s")
    wid = s * _NC + c
    is_max = wid < N_MAX_TILES
    scale = jnp.where(is_max, -1.0, 1.0).astype(jnp.float32)
    initv = jnp.where(is_max, 0.0, jnp.inf).astype(jnp.float32)
    init_splat = jnp.zeros((_L,), jnp.float32) + initv
    scale_splat = jnp.zeros((_L,), jnp.float32) + scale
    lanes = lax.iota(jnp.int32, _L)

    pltpu.sync_copy(offh_hbm.at[pl.ds(wid * OFF_NCH, OFF_NCH)], oh2d)
    pltpu.sync_copy(offt_hbm.at[pl.ds(wid * OFF_NCH, OFF_NCH)], ot2d)

    def zbody(r, carry):
        acc[pl.ds(r * _L, _L)] = init_splat
        return carry

    lax.fori_loop(0, ACC_R * DIM // _L, zbody, 0)

    pltpu.async_copy(off_hbm.at[ot2d.at[0]], rows0, sem0)

    def rmw_chunk(i, rows):
        isplat = jnp.zeros((_L,), jnp.int32) + i

        def rmw(j, carry2):
            jsplat = jnp.zeros((_L,), jnp.int32) + j
            hl = plsc.load_gather(oh2d, [isplat, jsplat])[0]
            abase = hl * DIM
            for k in range(DIM // _L):
                v = plsc.load_gather(rows, [jsplat, lanes + k * _L]) * scale_splat
                asl = pl.ds(abase + k * _L, _L)
                acc[asl] = jnp.minimum(acc[asl], v)
            return carry2

        lax.fori_loop(0, CH, rmw, 0)

    def pair(i2, carry):
        i = i2 * 2
        pltpu.async_copy(off_hbm.at[ot2d.at[i + 1]], rows1, sem1)
        pltpu.make_async_copy(off_hbm.at[ot2d.at[i]], rows0, sem0).wait()
        rmw_chunk(i, rows0)

        @pl.when(i2 < OFF_NCH // 2 - 1)
        def _():
            pltpu.async_copy(off_hbm.at[ot2d.at[i + 2]], rows0, sem0)

        pltpu.make_async_copy(off_hbm.at[ot2d.at[i + 1]], rows1, sem1).wait()
        rmw_chunk(i + 1, rows1)
        return carry

    lax.fori_loop(0, OFF_NCH // 2, pair, 0)
    pltpu.sync_copy(acc, out_hbm.at[pl.ds(wid * ACC_R * DIM, ACC_R * DIM)])


_BLK = 128


def _a2max_body(emb_ref, w1t_ref, w2t_ref, b1_ref, b2_ref, a2_ref, bmax_ref):
    x = emb_ref[...]
    a1 = jnp.maximum(jnp.dot(x, w1t_ref[...], preferred_element_type=jnp.float32) + b1_ref[...], 0.0)
    a2 = jnp.dot(a1, w2t_ref[...], preferred_element_type=jnp.float32) + b2_ref[...]
    a2_ref[...] = a2
    bmax_ref[...] = jnp.max(a2, axis=0, keepdims=True).reshape(1, 1, DIM)


def _make_a2max(nrows):
    nb = nrows // _BLK
    return pl.pallas_call(
        _a2max_body,
        grid=(nb,),
        in_specs=[
            pl.BlockSpec((_BLK, DIM), lambda i: (i, 0)),
            pl.BlockSpec((DIM, DIM), lambda i: (0, 0)),
            pl.BlockSpec((DIM, DIM), lambda i: (0, 0)),
            pl.BlockSpec((1, DIM), lambda i: (0, 0)),
            pl.BlockSpec((1, DIM), lambda i: (0, 0)),
        ],
        out_specs=[
            pl.BlockSpec((_BLK, DIM), lambda i: (i, 0)),
            pl.BlockSpec((1, 1, DIM), lambda i: (i, 0, 0)),
        ],
        out_shape=[
            jax.ShapeDtypeStruct((nrows, DIM), jnp.float32),
            jax.ShapeDtypeStruct((nb, 1, DIM), jnp.float32),
        ],
    )


def _uw_body(a2_ref, emb_ref, m_ref, uw_ref):
    w = jnp.exp(a2_ref[...] - m_ref[...])
    uw_ref[:, :DIM] = w * emb_ref[...]
    uw_ref[:, DIM:] = w


def _make_uw(nrows):
    return pl.pallas_call(
        _uw_body,
        grid=(nrows // _BLK,),
        in_specs=[
            pl.BlockSpec((_BLK, DIM), lambda i: (i, 0)),
            pl.BlockSpec((_BLK, DIM), lambda i: (i, 0)),
            pl.BlockSpec((1, DIM), lambda i: (0, 0)),
        ],
        out_specs=pl.BlockSpec((_BLK, 2 * DIM), lambda i: (i, 0)),
        out_shape=jax.ShapeDtypeStruct((nrows, 2 * DIM), jnp.float32),
    )


def _fin1_body(acc_ref, t_ref, out_ref):
    num = acc_ref[:, :DIM]
    den = acc_ref[:, DIM:]
    out_ref[...] = num / (den + 1e-16) * t_ref[...]


_tc_fin1 = pl.pallas_call(
    _fin1_body,
    grid=(SROWS // _BLK,),
    in_specs=[
        pl.BlockSpec((_BLK, 2 * DIM), lambda i: (i, 0)),
        pl.BlockSpec((_BLK, 1), lambda i: (i, 0)),
    ],
    out_specs=pl.BlockSpec((_BLK, DIM), lambda i: (i, 0)),
    out_shape=jax.ShapeDtypeStruct((SROWS, DIM), jnp.float32),
)


def _fin2_body(acc_ref, out_ref):
    agg = acc_ref[:, :DIM] / (acc_ref[:, DIM:] + 1e-16)
    nrm = jnp.sqrt(jnp.sum(agg * agg, axis=1, keepdims=True))
    out_ref[...] = agg / jnp.maximum(nrm, 1e-12)


_tc_fin2 = pl.pallas_call(
    _fin2_body,
    grid=(SROWS // _BLK,),
    in_specs=[pl.BlockSpec((_BLK, 2 * DIM), lambda i: (i, 0))],
    out_specs=pl.BlockSpec((_BLK, DIM), lambda i: (i, 0)),
    out_shape=jax.ShapeDtypeStruct((SROWS, DIM), jnp.float32),
)

N_PAD = _NW * BIN_W


def _finoff_body(x_ref, out_ref):
    i = pl.program_id(0)
    x = x_ref[...]
    rowid = i * _BLK + jax.lax.broadcasted_iota(jnp.int32, (_BLK, 1), 0)
    y = jnp.where(rowid < N_VISITS + N_CCSS, -x, jnp.where(jnp.isfinite(x), x, 0.0))
    out_ref[...] = jnp.maximum(y, 0.0)


_tc_finoff = pl.pallas_call(
    _finoff_body,
    grid=(N_PAD // _BLK,),
    in_specs=[pl.BlockSpec((_BLK, DIM), lambda i: (i, 0))],
    out_specs=pl.BlockSpec((_BLK, DIM), lambda i: (i, 0)),
    out_shape=jax.ShapeDtypeStruct((N_PAD, DIM), jnp.float32),
)


def _relu_body(x_ref, out_ref):
    out_ref[...] = jnp.maximum(x_ref[...], 0.0)


_tc_relu = pl.pallas_call(
    _relu_body,
    grid=(N_PAD // _BLK,),
    in_specs=[pl.BlockSpec((_BLK, DIM), lambda i: (i, 0))],
    out_specs=pl.BlockSpec((_BLK, DIM), lambda i: (i, 0)),
    out_shape=jax.ShapeDtypeStruct((N_PAD, DIM), jnp.float32),
)

_a2max_n = _make_a2max(N_PAD)
_a2max_v = _make_a2max(SROWS)
_uw_n = _make_uw(N_PAD)
_uw_v = _make_uw(SROWS)


def kernel(visit_emb, visit_offset, ccs_emb, ccs_offset, icd_emb, icd_offset, edge_index, visit_time, cW1, cb1, cW2, cb2, tW1, tb1, tW2, tb2):
    h = edge_index[0]
    t = edge_index[1]
    zpad = jnp.zeros((N_PAD - N_NODES, DIM), jnp.float32)
    embp = jnp.concatenate([visit_emb, ccs_emb, icd_emb, zpad], axis=0)
    offp = _tc_relu(jnp.concatenate([visit_offset, ccs_offset, icd_offset, zpad], axis=0))
    tt = (1.0 / visit_time).reshape(-1, 1)
    tt = jax.nn.relu(tt @ tW1.T + tb1)
    tt = tt @ tW2.T + tb2
    time_emb = jax.nn.softmax(tt, axis=0)
    time_p = jnp.concatenate([time_emb, jnp.zeros((SROWS - N_VISITS, 1), jnp.float32)], axis=0)
    w1t = cW1.T
    w2t = cW2.T
    b1r = cb1.reshape(1, DIM)
    b2r = cb2.reshape(1, DIM)

    evh, evt, vvh, vvt, offh, offt = _classify(h, t)
    evh2 = evh.reshape(_NW * EV_NCH, CH)
    evt2 = evt.reshape(_NW * EV_NCH, CH)
    vvh2 = vvh.reshape(_NW * VV_NCH, CH)
    vvt2 = vvt.reshape(_NW * VV_NCH, CH)
    offh2 = offh.reshape(_NW * OFF_NCH, CH)
    offt2 = offt.reshape(_NW * OFF_NCH, CH)

    vpad = jnp.zeros((N_PAD - N_VISITS, DIM), jnp.float32)
    for _ in range(2):
        a2, bmax = _a2max_n(embp, w1t, w2t, b1r, b2r)
        M = jnp.max(bmax[:, 0, :], axis=0, keepdims=True)
        uw1 = _uw_n(a2, embp, M)
        acc1 = _seg_sum_ev(evh2, evt2, uw1).reshape(SROWS, 256)
        agg2 = _tc_fin1(acc1, time_p)
        a2b, bmax2 = _a2max_v(agg2, w1t, w2t, b1r, b2r)
        M2 = jnp.max(bmax2[:, 0, :], axis=0, keepdims=True)
        uw2 = _uw_v(a2b, agg2, M2)
        acc2 = _seg_sum_vv(vvh2, vvt2, uw2).reshape(SROWS, 256)
        agg = _tc_fin2(acc2)

        oacc = _offsets(offh2, offt2, offp)
        flat = oacc.reshape(_NW, ACC_R, DIM)[:, :BIN_W, :].reshape(N_PAD, DIM)
        offp = _tc_finoff(flat)
        embp = jnp.concatenate([agg[:N_VISITS], vpad], axis=0)
    return embp[:N_VISITS], offp[:N_VISITS]


# speedup vs baseline: 1.1165x; 1.1165x over previous
import functools

import jax
import jax.numpy as jnp
from jax import lax
from jax.experimental import pallas as pl
from jax.experimental.pallas import tpu as pltpu
from jax.experimental.pallas import tpu_sc as plsc

N_VISITS = 6000
N_CCSS = 2000
N_ICDS = 2000
N_NODES = N_VISITS + N_CCSS + N_ICDS
N_EDGES = 320000
DIM = 128

_NC, _NS, _L = 2, 16, 16
_NW = _NC * _NS
SROWS = 6016
SENT = 6000
CH = 80
EPW = N_EDGES // _NW
_ROWS_PER_TILE = SROWS // _NS

EV_LCAP = 240
VV_LCAP = 400
OFF_LCAP = 800
EV_T = _L * EV_LCAP
VV_T = _L * VV_LCAP
OFF_T = _L * OFF_LCAP
EV_NCH = EV_T // CH
VV_NCH = VV_T // CH
OFF_NCH = OFF_T // CH
EV_BW = SROWS // _NW
TRASH2 = EV_BW
ACC2_R = EV_BW + 20
BIN_W = 320
TRASH = BIN_W
ACC_R = BIN_W + 16
N_MAX_TILES = (N_VISITS + N_CCSS) // BIN_W
CH2 = 2000
N_CH2 = N_EDGES // CH2

_mesh = plsc.VectorSubcoreMesh(core_axis_name="c", subcore_axis_name="s")


@functools.partial(
    pl.kernel,
    out_type=(
        jax.ShapeDtypeStruct((_NW * EV_T,), jnp.int32),
        jax.ShapeDtypeStruct((_NW * EV_T,), jnp.int32),
        jax.ShapeDtypeStruct((_NW * VV_T,), jnp.int32),
        jax.ShapeDtypeStruct((_NW * VV_T,), jnp.int32),
        jax.ShapeDtypeStruct((_NW * OFF_T,), jnp.int32),
        jax.ShapeDtypeStruct((_NW * OFF_T,), jnp.int32),
        jax.ShapeDtypeStruct((_NW * 48,), jnp.int32),
    ),
    mesh=_mesh,
    compiler_params=pltpu.CompilerParams(needs_layout_passes=False),
    scratch_types=[
        pltpu.VMEM((CH2,), jnp.int32),
        pltpu.VMEM((CH2,), jnp.int32),
        pltpu.VMEM((CH2,), jnp.int32),
        pltpu.VMEM((CH2,), jnp.int32),
        pltpu.VMEM((EV_T + _L,), jnp.int32),
        pltpu.VMEM((EV_T + _L,), jnp.int32),
        pltpu.VMEM((VV_T + _L,), jnp.int32),
        pltpu.VMEM((VV_T + _L,), jnp.int32),
        pltpu.VMEM((OFF_T + _L,), jnp.int32),
        pltpu.VMEM((OFF_T + _L,), jnp.int32),
        pltpu.VMEM((4 * _L,), jnp.int32),
        pltpu.VMEM((48,), jnp.int32),
        pltpu.SemaphoreType.DMA,
        pltpu.SemaphoreType.DMA,
    ],
)
def _classify(h_hbm, t_hbm, evh_hbm, evt_hbm, vvh_hbm, vvt_hbm, offh_hbm, offt_hbm, cnt_hbm,
              hv0, tv0, hv1, tv1, evh, evt, vvh, vvt, offh, offt, st, cv, sem0, sem1):
    c = lax.axis_index("c")
    s = lax.axis_index("s")
    wid = s * _NC + c
    bin_lo = wid * BIN_W
    bin_hi = bin_lo + BIN_W
    ev_lo = wid * EV_BW
    ev_hi = ev_lo + EV_BW
    lanes = lax.iota(jnp.int32, _L)

    st[pl.ds(_L, _L)] = jnp.zeros((_L,), jnp.int32)
    st[pl.ds(2 * _L, _L)] = jnp.zeros((_L,), jnp.int32)
    st[pl.ds(3 * _L, _L)] = jnp.zeros((_L,), jnp.int32)

    def bin_groups(hv, tv):
        def group(g, carry):
            sl = pl.ds(pl.multiple_of(g * _L, _L), _L)
            hh = hv[sl]
            tt = tv[sl]

            def compact(slot, mask, ref_h, ref_t, lcap, dump, hval):
                cnt = st[pl.ds(slot * _L, _L)]
                ok = mask & (cnt < lcap)
                pos = jnp.where(ok, lanes * lcap + cnt, dump)
                plsc.store_scatter(ref_h, [pos], hval)
                plsc.store_scatter(ref_t, [pos], tt)
                st[pl.ds(slot * _L, _L)] = cnt + ok.astype(jnp.int32)

            m_own = (hh >= bin_lo) & (hh < bin_hi)
            compact(3, m_own, offh, offt, OFF_LCAP, OFF_T, hh - bin_lo)
            inr = (hh >= ev_lo) & (hh < ev_hi)
            hloc = hh - ev_lo
            compact(1, inr & (tt >= N_VISITS), evh, evt, EV_LCAP, EV_T, hloc)
            compact(2, inr & (tt < N_VISITS), vvh, vvt, VV_LCAP, VV_T, hloc)
            return carry

        lax.fori_loop(0, CH2 // _L, group, 0)

    pltpu.async_copy(h_hbm.at[pl.ds(0, CH2)], hv0, sem0)
    pltpu.async_copy(t_hbm.at[pl.ds(0, CH2)], tv0, sem0)

    def pair(i2, carry):
        i = i2 * 2
        pltpu.async_copy(h_hbm.at[pl.ds((i + 1) * CH2, CH2)], hv1, sem1)
        pltpu.async_copy(t_hbm.at[pl.ds((i + 1) * CH2, CH2)], tv1, sem1)
        pltpu.make_async_copy(h_hbm.at[pl.ds(i * CH2, CH2)], hv0, sem0).wait()
        pltpu.make_async_copy(t_hbm.at[pl.ds(i * CH2, CH2)], tv0, sem0).wait()
        bin_groups(hv0, tv0)

        @pl.when(i2 < N_CH2 // 2 - 1)
        def _():
            pltpu.async_copy(h_hbm.at[pl.ds((i + 2) * CH2, CH2)], hv0, sem0)
            pltpu.async_copy(t_hbm.at[pl.ds((i + 2) * CH2, CH2)], tv0, sem0)

        pltpu.make_async_copy(h_hbm.at[pl.ds((i + 1) * CH2, CH2)], hv1, sem1).wait()
        pltpu.make_async_copy(t_hbm.at[pl.ds((i + 1) * CH2, CH2)], tv1, sem1).wait()
        bin_groups(hv1, tv1)
        return carry

    lax.fori_loop(0, N_CH2 // 2, pair, 0)

    def fill(slot, ref_h, ref_t, lcap, dump, pad_h):
        cnt = st[pl.ds(slot * _L, _L)]
        st[pl.ds(0, _L)] = jnp.zeros((_L,), jnp.int32)

        def fbody(i, carry):
            iv = st[pl.ds(0, _L)]
            st[pl.ds(0, _L)] = iv + 1
            need = iv >= cnt
            pos = jnp.where(need, lanes * lcap + iv, dump)
            plsc.store_scatter(ref_h, [pos], pad_h)
            plsc.store_scatter(ref_t, [pos], (lanes * 251 + iv) & 4095)
            return carry

        lax.fori_loop(0, lcap, fbody, 0)

    fill(1, evh, evt, EV_LCAP, EV_T, jnp.zeros((_L,), jnp.int32) + TRASH2)
    fill(2, vvh, vvt, VV_LCAP, VV_T, jnp.zeros((_L,), jnp.int32) + TRASH2)
    fill(3, offh, offt, OFF_LCAP, OFF_T, jnp.zeros((_L,), jnp.int32) + TRASH)

    cv[pl.ds(0, _L)] = st[pl.ds(_L, _L)]
    cv[pl.ds(_L, _L)] = st[pl.ds(2 * _L, _L)]
    cv[pl.ds(2 * _L, _L)] = st[pl.ds(3 * _L, _L)]
    pltpu.sync_copy(cv, cnt_hbm.at[pl.ds(wid * 48, 48)])
    pltpu.sync_copy(evh.at[pl.ds(0, EV_T)], evh_hbm.at[pl.ds(wid * EV_T, EV_T)])
    pltpu.sync_copy(evt.at[pl.ds(0, EV_T)], evt_hbm.at[pl.ds(wid * EV_T, EV_T)])
    pltpu.sync_copy(vvh.at[pl.ds(0, VV_T)], vvh_hbm.at[pl.ds(wid * VV_T, VV_T)])
    pltpu.sync_copy(vvt.at[pl.ds(0, VV_T)], vvt_hbm.at[pl.ds(wid * VV_T, VV_T)])
    pltpu.sync_copy(offh.at[pl.ds(0, OFF_T)], offh_hbm.at[pl.ds(wid * OFF_T, OFF_T)])
    pltpu.sync_copy(offt.at[pl.ds(0, OFF_T)], offt_hbm.at[pl.ds(wid * OFF_T, OFF_T)])


def _make_seg_sum(n_chunks, lcap, slot_off):
    cpl = lcap // CH

    @functools.partial(
        pl.kernel,
        out_type=jax.ShapeDtypeStruct((SROWS * 256,), jnp.float32),
        mesh=_mesh,
        compiler_params=pltpu.CompilerParams(needs_layout_passes=False),
        scratch_types=[
            pltpu.VMEM((n_chunks, CH), jnp.int32),
            pltpu.VMEM((n_chunks, CH), jnp.int32),
            pltpu.VMEM((CH, 256), jnp.float32),
            pltpu.VMEM((CH, 256), jnp.float32),
            pltpu.VMEM((ACC2_R * 256,), jnp.float32),
            pltpu.VMEM((_L,), jnp.int32),
            pltpu.SemaphoreType.DMA,
            pltpu.SemaphoreType.DMA,
        ],
    )
    def seg_sum(lh_hbm, lt_hbm, cnt_hbm, tbl_hbm, out_hbm, hl2d, tl2d, rows0, rows1, acc, ucv, sem0, sem1):
        c = lax.axis_index("c")
        s = lax.axis_index("s")
        wid = s * _NC + c
        lanes = lax.iota(jnp.int32, _L)
        zsplat = jnp.zeros((_L,), jnp.float32)

        pltpu.sync_copy(cnt_hbm.at[pl.ds(wid * 48 + slot_off, _L)], ucv)
        ucv[pl.ds(0, _L)] = (ucv[pl.ds(0, _L)] + (CH - 1)) // CH
        pltpu.sync_copy(lh_hbm.at[pl.ds(wid * n_chunks, n_chunks)], hl2d)
        pltpu.sync_copy(lt_hbm.at[pl.ds(wid * n_chunks, n_chunks)], tl2d)

        def zbody(r, carry):
            acc[pl.ds(r * _L, _L)] = zsplat
            return carry

        lax.fori_loop(0, ACC2_R * 256 // _L, zbody, 0)

        def pred(i_):
            lane = i_ // cpl
            lsp = jnp.zeros((_L,), jnp.int32) + lane
            u = plsc.load_gather(ucv, [lsp])[0]
            return (i_ - lane * cpl) < u

        @pl.when(pred(0))
        def _():
            pltpu.async_copy(tbl_hbm.at[tl2d.at[0]], rows0, sem0)

        def rmw_chunk(i, rows):
            isplat = jnp.zeros((_L,), jnp.int32) + i

            def rmw(j, carry2):
                jsplat = jnp.zeros((_L,), jnp.int32) + j
                hl = plsc.load_gather(hl2d, [isplat, jsplat])[0]
                abase = hl * 256
                for k in range(256 // _L):
                    v = plsc.load_gather(rows, [jsplat, lanes + k * _L])
                    asl = pl.ds(abase + k * _L, _L)
                    acc[asl] = acc[asl] + v
                return carry2

            lax.fori_loop(0, CH, rmw, 0)

        def pair(i2, carry):
            i = i2 * 2

            @pl.when(pred(i + 1))
            def _():
                pltpu.async_copy(tbl_hbm.at[tl2d.at[i + 1]], rows1, sem1)

            @pl.when(pred(i))
            def _():
                pltpu.make_async_copy(tbl_hbm.at[tl2d.at[i]], rows0, sem0).wait()
                rmw_chunk(i, rows0)

            @pl.when((i2 < n_chunks // 2 - 1) & pred(i + 2))
            def _():
                pltpu.async_copy(tbl_hbm.at[tl2d.at[i + 2]], rows0, sem0)

            @pl.when(pred(i + 1))
            def _():
                pltpu.make_async_copy(tbl_hbm.at[tl2d.at[i + 1]], rows1, sem1).wait()
                rmw_chunk(i + 1, rows1)
            return carry

        lax.fori_loop(0, n_chunks // 2, pair, 0)
        pltpu.sync_copy(
            acc.at[pl.ds(0, EV_BW * 256)],
            out_hbm.at[pl.ds(wid * EV_BW * 256, EV_BW * 256)],
        )

    return seg_sum


_seg_sum_ev = _make_seg_sum(EV_NCH, EV_LCAP, 0)
_seg_sum_vv = _make_seg_sum(VV_NCH, VV_LCAP, _L)


@functools.partial(
    pl.kernel,
    out_type=jax.ShapeDtypeStruct((_NW * ACC_R * DIM,), jnp.float32),
    mesh=_mesh,
    compiler_params=pltpu.CompilerParams(needs_layout_passes=False),
    scratch_types=[
        pltpu.VMEM((OFF_NCH, CH), jnp.int32),
        pltpu.VMEM((OFF_NCH, CH), jnp.int32),
        pltpu.VMEM((CH, DIM), jnp.float32),
        pltpu.VMEM((CH, DIM), jnp.float32),
        pltpu.VMEM((ACC_R * DIM,), jnp.float32),
        pltpu.VMEM((_L,), jnp.int32),
        pltpu.SemaphoreType.DMA,
        pltpu.SemaphoreType.DMA,
    ],
)
def _offsets(offh_hbm, offt_hbm, cnt_hbm, off_hbm, out_hbm, oh2d, ot2d, rows0, rows1, acc, ucv, sem0, sem1):
    c = lax.axis_index("c")
    s = lax.axis_index("s")
    wid = s * _NC + c
    is_max = wid < N_MAX_TILES
    scale = jnp.where(is_max, -1.0, 1.0).astype(jnp.float32)
    initv = jnp.where(is_max, 0.0, jnp.inf).astype(jnp.float32)
    init_splat = jnp.zeros((_L,), jnp.float32) + initv
    scale_splat = jnp.zeros((_L,), jnp.float32) + scale
    lanes = lax.iota(jnp.int32, _L)
    cpl = OFF_LCAP // CH

    pltpu.sync_copy(cnt_hbm.at[pl.ds(wid * 48 + 2 * _L, _L)], ucv)
    ucv[pl.ds(0, _L)] = (ucv[pl.ds(0, _L)] + (CH - 1)) // CH
    pltpu.sync_copy(offh_hbm.at[pl.ds(wid * OFF_NCH, OFF_NCH)], oh2d)
    pltpu.sync_copy(offt_hbm.at[pl.ds(wid * OFF_NCH, OFF_NCH)], ot2d)

    def zbody(r, carry):
        acc[pl.ds(r * _L, _L)] = init_splat
        return carry

    lax.fori_loop(0, ACC_R * DIM // _L, zbody, 0)

    def pred(i_):
        lane = i_ // cpl
        lsp = jnp.zeros((_L,), jnp.int32) + lane
        u = plsc.load_gather(ucv, [lsp])[0]
        return (i_ - lane * cpl) < u

    @pl.when(pred(0))
    def _():
        pltpu.async_copy(off_hbm.at[ot2d.at[0]], rows0, sem0)

    def rmw_chunk(i, rows):
        isplat = jnp.zeros((_L,), jnp.int32) + i

        def rmw(j, carry2):
            jsplat = jnp.zeros((_L,), jnp.int32) + j
            hl = plsc.load_gather(oh2d, [isplat, jsplat])[0]
            abase = hl * DIM
            for k in range(DIM // _L):
                v = plsc.load_gather(rows, [jsplat, lanes + k * _L]) * scale_splat
                asl = pl.ds(abase + k * _L, _L)
                acc[asl] = jnp.minimum(acc[asl], v)
            return carry2

        lax.fori_loop(0, CH, rmw, 0)

    def pair(i2, carry):
        i = i2 * 2

        @pl.when(pred(i + 1))
        def _():
            pltpu.async_copy(off_hbm.at[ot2d.at[i + 1]], rows1, sem1)

        @pl.when(pred(i))
        def _():
            pltpu.make_async_copy(off_hbm.at[ot2d.at[i]], rows0, sem0).wait()
            rmw_chunk(i, rows0)

        @pl.when((i2 < OFF_NCH // 2 - 1) & pred(i + 2))
        def _():
            pltpu.async_copy(off_hbm.at[ot2d.at[i + 2]], rows0, sem0)

        @pl.when(pred(i + 1))
        def _():
            pltpu.make_async_copy(off_hbm.at[ot2d.at[i + 1]], rows1, sem1).wait()
            rmw_chunk(i + 1, rows1)
        return carry

    lax.fori_loop(0, OFF_NCH // 2, pair, 0)
    pltpu.sync_copy(acc, out_hbm.at[pl.ds(wid * ACC_R * DIM, ACC_R * DIM)])


_BLK = 128


def _a2max_body(emb_ref, w1t_ref, w2t_ref, b1_ref, b2_ref, a2_ref, bmax_ref):
    x = emb_ref[...]
    a1 = jnp.maximum(jnp.dot(x, w1t_ref[...], preferred_element_type=jnp.float32) + b1_ref[...], 0.0)
    a2 = jnp.dot(a1, w2t_ref[...], preferred_element_type=jnp.float32) + b2_ref[...]
    a2_ref[...] = a2
    bmax_ref[...] = jnp.max(a2, axis=0, keepdims=True).reshape(1, 1, DIM)


def _make_a2max(nrows):
    nb = nrows // _BLK
    return pl.pallas_call(
        _a2max_body,
        grid=(nb,),
        in_specs=[
            pl.BlockSpec((_BLK, DIM), lambda i: (i, 0)),
            pl.BlockSpec((DIM, DIM), lambda i: (0, 0)),
            pl.BlockSpec((DIM, DIM), lambda i: (0, 0)),
            pl.BlockSpec((1, DIM), lambda i: (0, 0)),
            pl.BlockSpec((1, DIM), lambda i: (0, 0)),
        ],
        out_specs=[
            pl.BlockSpec((_BLK, DIM), lambda i: (i, 0)),
            pl.BlockSpec((1, 1, DIM), lambda i: (i, 0, 0)),
        ],
        out_shape=[
            jax.ShapeDtypeStruct((nrows, DIM), jnp.float32),
            jax.ShapeDtypeStruct((nb, 1, DIM), jnp.float32),
        ],
    )


def _uw_body(a2_ref, emb_ref, m_ref, uw_ref):
    w = jnp.exp(a2_ref[...] - m_ref[...])
    uw_ref[:, :DIM] = w * emb_ref[...]
    uw_ref[:, DIM:] = w


def _make_uw(nrows):
    return pl.pallas_call(
        _uw_body,
        grid=(nrows // _BLK,),
        in_specs=[
            pl.BlockSpec((_BLK, DIM), lambda i: (i, 0)),
            pl.BlockSpec((_BLK, DIM), lambda i: (i, 0)),
            pl.BlockSpec((1, DIM), lambda i: (0, 0)),
        ],
        out_specs=pl.BlockSpec((_BLK, 2 * DIM), lambda i: (i, 0)),
        out_shape=jax.ShapeDtypeStruct((nrows, 2 * DIM), jnp.float32),
    )


def _fin1_body(acc_ref, t_ref, out_ref):
    num = acc_ref[:, :DIM]
    den = acc_ref[:, DIM:]
    out_ref[...] = num / (den + 1e-16) * t_ref[...]


_tc_fin1 = pl.pallas_call(
    _fin1_body,
    grid=(SROWS // _BLK,),
    in_specs=[
        pl.BlockSpec((_BLK, 2 * DIM), lambda i: (i, 0)),
        pl.BlockSpec((_BLK, 1), lambda i: (i, 0)),
    ],
    out_specs=pl.BlockSpec((_BLK, DIM), lambda i: (i, 0)),
    out_shape=jax.ShapeDtypeStruct((SROWS, DIM), jnp.float32),
)


def _fin2_body(acc_ref, out_ref):
    agg = acc_ref[:, :DIM] / (acc_ref[:, DIM:] + 1e-16)
    nrm = jnp.sqrt(jnp.sum(agg * agg, axis=1, keepdims=True))
    out_ref[...] = agg / jnp.maximum(nrm, 1e-12)


_tc_fin2 = pl.pallas_call(
    _fin2_body,
    grid=(SROWS // _BLK,),
    in_specs=[pl.BlockSpec((_BLK, 2 * DIM), lambda i: (i, 0))],
    out_specs=pl.BlockSpec((_BLK, DIM), lambda i: (i, 0)),
    out_shape=jax.ShapeDtypeStruct((SROWS, DIM), jnp.float32),
)

N_PAD = _NW * BIN_W


def _finoff_body(x_ref, out_ref):
    i = pl.program_id(0)
    x = x_ref[...]
    rowid = i * _BLK + jax.lax.broadcasted_iota(jnp.int32, (_BLK, 1), 0)
    y = jnp.where(rowid < N_VISITS + N_CCSS, -x, jnp.where(jnp.isfinite(x), x, 0.0))
    out_ref[...] = jnp.maximum(y, 0.0)


_tc_finoff = pl.pallas_call(
    _finoff_body,
    grid=(N_PAD // _BLK,),
    in_specs=[pl.BlockSpec((_BLK, DIM), lambda i: (i, 0))],
    out_specs=pl.BlockSpec((_BLK, DIM), lambda i: (i, 0)),
    out_shape=jax.ShapeDtypeStruct((N_PAD, DIM), jnp.float32),
)


def _relu_body(x_ref, out_ref):
    out_ref[...] = jnp.maximum(x_ref[...], 0.0)


_tc_relu = pl.pallas_call(
    _relu_body,
    grid=(N_PAD // _BLK,),
    in_specs=[pl.BlockSpec((_BLK, DIM), lambda i: (i, 0))],
    out_specs=pl.BlockSpec((_BLK, DIM), lambda i: (i, 0)),
    out_shape=jax.ShapeDtypeStruct((N_PAD, DIM), jnp.float32),
)

_a2max_n = _make_a2max(N_PAD)
_a2max_v = _make_a2max(SROWS)
_uw_n = _make_uw(N_PAD)
_uw_v = _make_uw(SROWS)


def kernel(visit_emb, visit_offset, ccs_emb, ccs_offset, icd_emb, icd_offset, edge_index, visit_time, cW1, cb1, cW2, cb2, tW1, tb1, tW2, tb2):
    h = edge_index[0]
    t = edge_index[1]
    zpad = jnp.zeros((N_PAD - N_NODES, DIM), jnp.float32)
    embp = jnp.concatenate([visit_emb, ccs_emb, icd_emb, zpad], axis=0)
    offp = _tc_relu(jnp.concatenate([visit_offset, ccs_offset, icd_offset, zpad], axis=0))
    tt = (1.0 / visit_time).reshape(-1, 1)
    tt = jax.nn.relu(tt @ tW1.T + tb1)
    tt = tt @ tW2.T + tb2
    time_emb = jax.nn.softmax(tt, axis=0)
    time_p = jnp.concatenate([time_emb, jnp.zeros((SROWS - N_VISITS, 1), jnp.float32)], axis=0)
    w1t = cW1.T
    w2t = cW2.T
    b1r = cb1.reshape(1, DIM)
    b2r = cb2.reshape(1, DIM)

    evh, evt, vvh, vvt, offh, offt, cnts = _classify(h, t)
    evh2 = evh.reshape(_NW * EV_NCH, CH)
    evt2 = evt.reshape(_NW * EV_NCH, CH)
    vvh2 = vvh.reshape(_NW * VV_NCH, CH)
    vvt2 = vvt.reshape(_NW * VV_NCH, CH)
    offh2 = offh.reshape(_NW * OFF_NCH, CH)
    offt2 = offt.reshape(_NW * OFF_NCH, CH)

    vpad = jnp.zeros((N_PAD - N_VISITS, DIM), jnp.float32)
    for _ in range(2):
        a2, bmax = _a2max_n(embp, w1t, w2t, b1r, b2r)
        M = jnp.max(bmax[:, 0, :], axis=0, keepdims=True)
        uw1 = _uw_n(a2, embp, M)
        acc1 = _seg_sum_ev(evh2, evt2, cnts, uw1).reshape(SROWS, 256)
        agg2 = _tc_fin1(acc1, time_p)
        a2b, bmax2 = _a2max_v(agg2, w1t, w2t, b1r, b2r)
        M2 = jnp.max(bmax2[:, 0, :], axis=0, keepdims=True)
        uw2 = _uw_v(a2b, agg2, M2)
        acc2 = _seg_sum_vv(vvh2, vvt2, cnts, uw2).reshape(SROWS, 256)
        agg = _tc_fin2(acc2)

        oacc = _offsets(offh2, offt2, cnts, offp)
        flat = oacc.reshape(_NW, ACC_R, DIM)[:, :BIN_W, :].reshape(N_PAD, DIM)
        offp = _tc_finoff(flat)
        embp = jnp.concatenate([agg[:N_VISITS], vpad], axis=0)
    return embp[:N_VISITS], offp[:N_VISITS]


# speedup vs baseline: 1.1251x; 1.0077x over previous
import functools

import jax
import jax.numpy as jnp
from jax import lax
from jax.experimental import pallas as pl
from jax.experimental.pallas import tpu as pltpu
from jax.experimental.pallas import tpu_sc as plsc

N_VISITS = 6000
N_CCSS = 2000
N_ICDS = 2000
N_NODES = N_VISITS + N_CCSS + N_ICDS
N_EDGES = 320000
DIM = 128

_NC, _NS, _L = 2, 16, 16
_NW = _NC * _NS
SROWS = 6016
SENT = 6000
CH = 80
EPW = N_EDGES // _NW
_ROWS_PER_TILE = SROWS // _NS

EV_LCAP = 240
VV_LCAP = 400
OFF_LCAP = 800
EV_T = _L * EV_LCAP
VV_T = _L * VV_LCAP
OFF_T = _L * OFF_LCAP
EV_NCH = EV_T // CH
VV_NCH = VV_T // CH
OFF_NCH = OFF_T // CH
EV_BW = SROWS // _NW
TRASH2 = EV_BW
ACC2_R = EV_BW + 20
BIN_W = 320
TRASH = BIN_W
ACC_R = BIN_W + 16
N_MAX_TILES = (N_VISITS + N_CCSS) // BIN_W
CH2 = 2000
N_CH2 = N_EDGES // CH2

_mesh = plsc.VectorSubcoreMesh(core_axis_name="c", subcore_axis_name="s")


@functools.partial(
    pl.kernel,
    out_type=(
        jax.ShapeDtypeStruct((_NW * EV_T,), jnp.int32),
        jax.ShapeDtypeStruct((_NW * EV_T,), jnp.int32),
        jax.ShapeDtypeStruct((_NW * VV_T,), jnp.int32),
        jax.ShapeDtypeStruct((_NW * VV_T,), jnp.int32),
        jax.ShapeDtypeStruct((_NW * OFF_T,), jnp.int32),
        jax.ShapeDtypeStruct((_NW * OFF_T,), jnp.int32),
        jax.ShapeDtypeStruct((_NW * 48,), jnp.int32),
    ),
    mesh=_mesh,
    compiler_params=pltpu.CompilerParams(needs_layout_passes=False),
    scratch_types=[
        pltpu.VMEM((CH2,), jnp.int32),
        pltpu.VMEM((CH2,), jnp.int32),
        pltpu.VMEM((CH2,), jnp.int32),
        pltpu.VMEM((CH2,), jnp.int32),
        pltpu.VMEM((EV_T + _L,), jnp.int32),
        pltpu.VMEM((EV_T + _L,), jnp.int32),
        pltpu.VMEM((VV_T + _L,), jnp.int32),
        pltpu.VMEM((VV_T + _L,), jnp.int32),
        pltpu.VMEM((OFF_T + _L,), jnp.int32),
        pltpu.VMEM((OFF_T + _L,), jnp.int32),
        pltpu.VMEM((4 * _L,), jnp.int32),
        pltpu.VMEM((48,), jnp.int32),
        pltpu.SemaphoreType.DMA,
        pltpu.SemaphoreType.DMA,
    ],
)
def _classify(h_hbm, t_hbm, evh_hbm, evt_hbm, vvh_hbm, vvt_hbm, offh_hbm, offt_hbm, cnt_hbm,
              hv0, tv0, hv1, tv1, evh, evt, vvh, vvt, offh, offt, st, cv, sem0, sem1):
    c = lax.axis_index("c")
    s = lax.axis_index("s")
    wid = s * _NC + c
    bin_lo = wid * BIN_W
    bin_hi = bin_lo + BIN_W
    ev_lo = wid * EV_BW
    ev_hi = ev_lo + EV_BW
    lanes = lax.iota(jnp.int32, _L)

    st[pl.ds(_L, _L)] = jnp.zeros((_L,), jnp.int32)
    st[pl.ds(2 * _L, _L)] = jnp.zeros((_L,), jnp.int32)
    st[pl.ds(3 * _L, _L)] = jnp.zeros((_L,), jnp.int32)

    def bin_groups(hv, tv):
        def compact(slot, mask, ref_h, ref_t, lcap, dump, hval, tt):
            cnt = st[pl.ds(slot * _L, _L)]
            ok = mask & (cnt < lcap)
            pos = jnp.where(ok, lanes * lcap + cnt, dump)
            plsc.store_scatter(ref_h, [pos], hval)
            plsc.store_scatter(ref_t, [pos], tt)
            st[pl.ds(slot * _L, _L)] = cnt + ok.astype(jnp.int32)

        def group(g4, carry):
            for u in range(4):
                g = g4 * 4 + u
                sl = pl.ds(pl.multiple_of(g * _L, _L), _L)
                hh = hv[sl]
                tt = tv[sl]
                m_own = (hh >= bin_lo) & (hh < bin_hi)
                compact(3, m_own, offh, offt, OFF_LCAP, OFF_T, hh - bin_lo, tt)
                inr = (hh >= ev_lo) & (hh < ev_hi)
                hloc = hh - ev_lo
                compact(1, inr & (tt >= N_VISITS), evh, evt, EV_LCAP, EV_T, hloc, tt)
                compact(2, inr & (tt < N_VISITS), vvh, vvt, VV_LCAP, VV_T, hloc, tt)
            return carry

        lax.fori_loop(0, CH2 // _L // 4, group, 0)

    pltpu.async_copy(h_hbm.at[pl.ds(0, CH2)], hv0, sem0)
    pltpu.async_copy(t_hbm.at[pl.ds(0, CH2)], tv0, sem0)

    def pair(i2, carry):
        i = i2 * 2
        pltpu.async_copy(h_hbm.at[pl.ds((i + 1) * CH2, CH2)], hv1, sem1)
        pltpu.async_copy(t_hbm.at[pl.ds((i + 1) * CH2, CH2)], tv1, sem1)
        pltpu.make_async_copy(h_hbm.at[pl.ds(i * CH2, CH2)], hv0, sem0).wait()
        pltpu.make_async_copy(t_hbm.at[pl.ds(i * CH2, CH2)], tv0, sem0).wait()
        bin_groups(hv0, tv0)

        @pl.when(i2 < N_CH2 // 2 - 1)
        def _():
            pltpu.async_copy(h_hbm.at[pl.ds((i + 2) * CH2, CH2)], hv0, sem0)
            pltpu.async_copy(t_hbm.at[pl.ds((i + 2) * CH2, CH2)], tv0, sem0)

        pltpu.make_async_copy(h_hbm.at[pl.ds((i + 1) * CH2, CH2)], hv1, sem1).wait()
        pltpu.make_async_copy(t_hbm.at[pl.ds((i + 1) * CH2, CH2)], tv1, sem1).wait()
        bin_groups(hv1, tv1)
        return carry

    lax.fori_loop(0, N_CH2 // 2, pair, 0)

    def fill(slot, ref_h, ref_t, lcap, dump, pad_h):
        cnt = st[pl.ds(slot * _L, _L)]
        st[pl.ds(0, _L)] = jnp.zeros((_L,), jnp.int32)

        def fbody(i, carry):
            iv = st[pl.ds(0, _L)]
            st[pl.ds(0, _L)] = iv + 1
            need = iv >= cnt
            pos = jnp.where(need, lanes * lcap + iv, dump)
            plsc.store_scatter(ref_h, [pos], pad_h)
            plsc.store_scatter(ref_t, [pos], (lanes * 251 + iv) & 4095)
            return carry

        lax.fori_loop(0, lcap, fbody, 0)

    fill(1, evh, evt, EV_LCAP, EV_T, jnp.zeros((_L,), jnp.int32) + TRASH2)
    fill(2, vvh, vvt, VV_LCAP, VV_T, jnp.zeros((_L,), jnp.int32) + TRASH2)
    fill(3, offh, offt, OFF_LCAP, OFF_T, jnp.zeros((_L,), jnp.int32) + TRASH)

    cv[pl.ds(0, _L)] = st[pl.ds(_L, _L)]
    cv[pl.ds(_L, _L)] = st[pl.ds(2 * _L, _L)]
    cv[pl.ds(2 * _L, _L)] = st[pl.ds(3 * _L, _L)]
    pltpu.sync_copy(cv, cnt_hbm.at[pl.ds(wid * 48, 48)])
    pltpu.sync_copy(evh.at[pl.ds(0, EV_T)], evh_hbm.at[pl.ds(wid * EV_T, EV_T)])
    pltpu.sync_copy(evt.at[pl.ds(0, EV_T)], evt_hbm.at[pl.ds(wid * EV_T, EV_T)])
    pltpu.sync_copy(vvh.at[pl.ds(0, VV_T)], vvh_hbm.at[pl.ds(wid * VV_T, VV_T)])
    pltpu.sync_copy(vvt.at[pl.ds(0, VV_T)], vvt_hbm.at[pl.ds(wid * VV_T, VV_T)])
    pltpu.sync_copy(offh.at[pl.ds(0, OFF_T)], offh_hbm.at[pl.ds(wid * OFF_T, OFF_T)])
    pltpu.sync_copy(offt.at[pl.ds(0, OFF_T)], offt_hbm.at[pl.ds(wid * OFF_T, OFF_T)])


def _make_seg_sum(n_chunks, lcap, slot_off):
    cpl = lcap // CH

    @functools.partial(
        pl.kernel,
        out_type=jax.ShapeDtypeStruct((SROWS * 256,), jnp.float32),
        mesh=_mesh,
        compiler_params=pltpu.CompilerParams(needs_layout_passes=False),
        scratch_types=[
            pltpu.VMEM((n_chunks, CH), jnp.int32),
            pltpu.VMEM((n_chunks, CH), jnp.int32),
            pltpu.VMEM((CH, 256), jnp.float32),
            pltpu.VMEM((CH, 256), jnp.float32),
            pltpu.VMEM((ACC2_R * 256,), jnp.float32),
            pltpu.VMEM((_L,), jnp.int32),
            pltpu.SemaphoreType.DMA,
            pltpu.SemaphoreType.DMA,
        ],
    )
    def seg_sum(lh_hbm, lt_hbm, cnt_hbm, tbl_hbm, out_hbm, hl2d, tl2d, rows0, rows1, acc, ucv, sem0, sem1):
        c = lax.axis_index("c")
        s = lax.axis_index("s")
        wid = s * _NC + c
        lanes = lax.iota(jnp.int32, _L)
        zsplat = jnp.zeros((_L,), jnp.float32)

        pltpu.sync_copy(cnt_hbm.at[pl.ds(wid * 48 + slot_off, _L)], ucv)
        ucv[pl.ds(0, _L)] = (ucv[pl.ds(0, _L)] + (CH - 1)) // CH
        pltpu.sync_copy(lh_hbm.at[pl.ds(wid * n_chunks, n_chunks)], hl2d)
        pltpu.sync_copy(lt_hbm.at[pl.ds(wid * n_chunks, n_chunks)], tl2d)

        def zbody(r, carry):
            acc[pl.ds(r * _L, _L)] = zsplat
            return carry

        lax.fori_loop(0, ACC2_R * 256 // _L, zbody, 0)

        def pred(i_):
            lane = i_ // cpl
            lsp = jnp.zeros((_L,), jnp.int32) + lane
            u = plsc.load_gather(ucv, [lsp])[0]
            return (i_ - lane * cpl) < u

        @pl.when(pred(0))
        def _():
            pltpu.async_copy(tbl_hbm.at[tl2d.at[0]], rows0, sem0)

        def rmw_chunk(i, rows):
            isplat = jnp.zeros((_L,), jnp.int32) + i

            def rmw(j, carry2):
                jsplat = jnp.zeros((_L,), jnp.int32) + j
                hl = plsc.load_gather(hl2d, [isplat, jsplat])[0]
                abase = hl * 256
                for k in range(256 // _L):
                    v = plsc.load_gather(rows, [jsplat, lanes + k * _L])
                    asl = pl.ds(abase + k * _L, _L)
                    acc[asl] = acc[asl] + v
                return carry2

            lax.fori_loop(0, CH, rmw, 0)

        def pair(i2, carry):
            i = i2 * 2

            @pl.when(pred(i + 1))
            def _():
                pltpu.async_copy(tbl_hbm.at[tl2d.at[i + 1]], rows1, sem1)

            @pl.when(pred(i))
            def _():
                pltpu.make_async_copy(tbl_hbm.at[tl2d.at[i]], rows0, sem0).wait()
                rmw_chunk(i, rows0)

            @pl.when((i2 < n_chunks // 2 - 1) & pred(i + 2))
            def _():
                pltpu.async_copy(tbl_hbm.at[tl2d.at[i + 2]], rows0, sem0)

            @pl.when(pred(i + 1))
            def _():
                pltpu.make_async_copy(tbl_hbm.at[tl2d.at[i + 1]], rows1, sem1).wait()
                rmw_chunk(i + 1, rows1)
            return carry

        lax.fori_loop(0, n_chunks // 2, pair, 0)
        pltpu.sync_copy(
            acc.at[pl.ds(0, EV_BW * 256)],
            out_hbm.at[pl.ds(wid * EV_BW * 256, EV_BW * 256)],
        )

    return seg_sum


_seg_sum_ev = _make_seg_sum(EV_NCH, EV_LCAP, 0)
_seg_sum_vv = _make_seg_sum(VV_NCH, VV_LCAP, _L)


@functools.partial(
    pl.kernel,
    out_type=jax.ShapeDtypeStruct((_NW * ACC_R * DIM,), jnp.float32),
    mesh=_mesh,
    compiler_params=pltpu.CompilerParams(needs_layout_passes=False),
    scratch_types=[
        pltpu.VMEM((OFF_NCH, CH), jnp.int32),
        pltpu.VMEM((OFF_NCH, CH), jnp.int32),
        pltpu.VMEM((CH, DIM), jnp.float32),
        pltpu.VMEM((CH, DIM), jnp.float32),
        pltpu.VMEM((ACC_R * DIM,), jnp.float32),
        pltpu.VMEM((_L,), jnp.int32),
        pltpu.SemaphoreType.DMA,
        pltpu.SemaphoreType.DMA,
    ],
)
def _offsets(offh_hbm, offt_hbm, cnt_hbm, off_hbm, out_hbm, oh2d, ot2d, rows0, rows1, acc, ucv, sem0, sem1):
    c = lax.axis_index("c")
    s = lax.axis_index("s")
    wid = s * _NC + c
    is_max = wid < N_MAX_TILES
    scale = jnp.where(is_max, -1.0, 1.0).astype(jnp.float32)
    initv = jnp.where(is_max, 0.0, jnp.inf).astype(jnp.float32)
    init_splat = jnp.zeros((_L,), jnp.float32) + initv
    scale_splat = jnp.zeros((_L,), jnp.float32) + scale
    lanes = lax.iota(jnp.int32, _L)
    cpl = OFF_LCAP // CH

    pltpu.sync_copy(cnt_hbm.at[pl.ds(wid * 48 + 2 * _L, _L)], ucv)
    ucv[pl.ds(0, _L)] = (ucv[pl.ds(0, _L)] + (CH - 1)) // CH
    pltpu.sync_copy(offh_hbm.at[pl.ds(wid * OFF_NCH, OFF_NCH)], oh2d)
    pltpu.sync_copy(offt_hbm.at[pl.ds(wid * OFF_NCH, OFF_NCH)], ot2d)

    def zbody(r, carry):
        acc[pl.ds(r * _L, _L)] = init_splat
        return carry

    lax.fori_loop(0, ACC_R * DIM // _L, zbody, 0)

    def pred(i_):
        lane = i_ // cpl
        lsp = jnp.zeros((_L,), jnp.int32) + lane
        u = plsc.load_gather(ucv, [lsp])[0]
        return (i_ - lane * cpl) < u

    @pl.when(pred(0))
    def _():
        pltpu.async_copy(off_hbm.at[ot2d.at[0]], rows0, sem0)

    def rmw_chunk(i, rows):
        isplat = jnp.zeros((_L,), jnp.int32) + i

        def rmw(j, carry2):
            jsplat = jnp.zeros((_L,), jnp.int32) + j
            hl = plsc.load_gather(oh2d, [isplat, jsplat])[0]
            abase = hl * DIM
            for k in range(DIM // _L):
                v = plsc.load_gather(rows, [jsplat, lanes + k * _L]) * scale_splat
                asl = pl.ds(abase + k * _L, _L)
                acc[asl] = jnp.minimum(acc[asl], v)
            return carry2

        lax.fori_loop(0, CH, rmw, 0)

    def pair(i2, carry):
        i = i2 * 2

        @pl.when(pred(i + 1))
        def _():
            pltpu.async_copy(off_hbm.at[ot2d.at[i + 1]], rows1, sem1)

        @pl.when(pred(i))
        def _():
            pltpu.make_async_copy(off_hbm.at[ot2d.at[i]], rows0, sem0).wait()
            rmw_chunk(i, rows0)

        @pl.when((i2 < OFF_NCH // 2 - 1) & pred(i + 2))
        def _():
            pltpu.async_copy(off_hbm.at[ot2d.at[i + 2]], rows0, sem0)

        @pl.when(pred(i + 1))
        def _():
            pltpu.make_async_copy(off_hbm.at[ot2d.at[i + 1]], rows1, sem1).wait()
            rmw_chunk(i + 1, rows1)
        return carry

    lax.fori_loop(0, OFF_NCH // 2, pair, 0)
    pltpu.sync_copy(acc, out_hbm.at[pl.ds(wid * ACC_R * DIM, ACC_R * DIM)])


_BLK = 128


def _a2max_body(emb_ref, w1t_ref, w2t_ref, b1_ref, b2_ref, a2_ref, bmax_ref):
    x = emb_ref[...]
    a1 = jnp.maximum(jnp.dot(x, w1t_ref[...], preferred_element_type=jnp.float32) + b1_ref[...], 0.0)
    a2 = jnp.dot(a1, w2t_ref[...], preferred_element_type=jnp.float32) + b2_ref[...]
    a2_ref[...] = a2
    bmax_ref[...] = jnp.max(a2, axis=0, keepdims=True).reshape(1, 1, DIM)


def _make_a2max(nrows):
    nb = nrows // _BLK
    return pl.pallas_call(
        _a2max_body,
        grid=(nb,),
        in_specs=[
            pl.BlockSpec((_BLK, DIM), lambda i: (i, 0)),
            pl.BlockSpec((DIM, DIM), lambda i: (0, 0)),
            pl.BlockSpec((DIM, DIM), lambda i: (0, 0)),
            pl.BlockSpec((1, DIM), lambda i: (0, 0)),
            pl.BlockSpec((1, DIM), lambda i: (0, 0)),
        ],
        out_specs=[
            pl.BlockSpec((_BLK, DIM), lambda i: (i, 0)),
            pl.BlockSpec((1, 1, DIM), lambda i: (i, 0, 0)),
        ],
        out_shape=[
            jax.ShapeDtypeStruct((nrows, DIM), jnp.float32),
            jax.ShapeDtypeStruct((nb, 1, DIM), jnp.float32),
        ],
    )


def _uw_body(a2_ref, emb_ref, m_ref, uw_ref):
    w = jnp.exp(a2_ref[...] - m_ref[...])
    uw_ref[:, :DIM] = w * emb_ref[...]
    uw_ref[:, DIM:] = w


def _make_uw(nrows):
    return pl.pallas_call(
        _uw_body,
        grid=(nrows // _BLK,),
        in_specs=[
            pl.BlockSpec((_BLK, DIM), lambda i: (i, 0)),
            pl.BlockSpec((_BLK, DIM), lambda i: (i, 0)),
            pl.BlockSpec((1, DIM), lambda i: (0, 0)),
        ],
        out_specs=pl.BlockSpec((_BLK, 2 * DIM), lambda i: (i, 0)),
        out_shape=jax.ShapeDtypeStruct((nrows, 2 * DIM), jnp.float32),
    )


def _fin1_body(acc_ref, t_ref, out_ref):
    num = acc_ref[:, :DIM]
    den = acc_ref[:, DIM:]
    out_ref[...] = num / (den + 1e-16) * t_ref[...]


_tc_fin1 = pl.pallas_call(
    _fin1_body,
    grid=(SROWS // _BLK,),
    in_specs=[
        pl.BlockSpec((_BLK, 2 * DIM), lambda i: (i, 0)),
        pl.BlockSpec((_BLK, 1), lambda i: (i, 0)),
    ],
    out_specs=pl.BlockSpec((_BLK, DIM), lambda i: (i, 0)),
    out_shape=jax.ShapeDtypeStruct((SROWS, DIM), jnp.float32),
)


def _fin2_body(acc_ref, out_ref):
    agg = acc_ref[:, :DIM] / (acc_ref[:, DIM:] + 1e-16)
    nrm = jnp.sqrt(jnp.sum(agg * agg, axis=1, keepdims=True))
    out_ref[...] = agg / jnp.maximum(nrm, 1e-12)


_tc_fin2 = pl.pallas_call(
    _fin2_body,
    grid=(SROWS // _BLK,),
    in_specs=[pl.BlockSpec((_BLK, 2 * DIM), lambda i: (i, 0))],
    out_specs=pl.BlockSpec((_BLK, DIM), lambda i: (i, 0)),
    out_shape=jax.ShapeDtypeStruct((SROWS, DIM), jnp.float32),
)

N_PAD = _NW * BIN_W


def _finoff_body(x_ref, out_ref):
    i = pl.program_id(0)
    x = x_ref[...]
    rowid = i * _BLK + jax.lax.broadcasted_iota(jnp.int32, (_BLK, 1), 0)
    y = jnp.where(rowid < N_VISITS + N_CCSS, -x, jnp.where(jnp.isfinite(x), x, 0.0))
    out_ref[...] = jnp.maximum(y, 0.0)


_tc_finoff = pl.pallas_call(
    _finoff_body,
    grid=(N_PAD // _BLK,),
    in_specs=[pl.BlockSpec((_BLK, DIM), lambda i: (i, 0))],
    out_specs=pl.BlockSpec((_BLK, DIM), lambda i: (i, 0)),
    out_shape=jax.ShapeDtypeStruct((N_PAD, DIM), jnp.float32),
)


def _relu_body(x_ref, out_ref):
    out_ref[...] = jnp.maximum(x_ref[...], 0.0)


_tc_relu = pl.pallas_call(
    _relu_body,
    grid=(N_PAD // _BLK,),
    in_specs=[pl.BlockSpec((_BLK, DIM), lambda i: (i, 0))],
    out_specs=pl.BlockSpec((_BLK, DIM), lambda i: (i, 0)),
    out_shape=jax.ShapeDtypeStruct((N_PAD, DIM), jnp.float32),
)

_a2max_n = _make_a2max(N_PAD)
_a2max_v = _make_a2max(SROWS)
_uw_n = _make_uw(N_PAD)
_uw_v = _make_uw(SROWS)


def kernel(visit_emb, visit_offset, ccs_emb, ccs_offset, icd_emb, icd_offset, edge_index, visit_time, cW1, cb1, cW2, cb2, tW1, tb1, tW2, tb2):
    h = edge_index[0]
    t = edge_index[1]
    zpad = jnp.zeros((N_PAD - N_NODES, DIM), jnp.float32)
    embp = jnp.concatenate([visit_emb, ccs_emb, icd_emb, zpad], axis=0)
    offp = _tc_relu(jnp.concatenate([visit_offset, ccs_offset, icd_offset, zpad], axis=0))
    tt = (1.0 / visit_time).reshape(-1, 1)
    tt = jax.nn.relu(tt @ tW1.T + tb1)
    tt = tt @ tW2.T + tb2
    time_emb = jax.nn.softmax(tt, axis=0)
    time_p = jnp.concatenate([time_emb, jnp.zeros((SROWS - N_VISITS, 1), jnp.float32)], axis=0)
    w1t = cW1.T
    w2t = cW2.T
    b1r = cb1.reshape(1, DIM)
    b2r = cb2.reshape(1, DIM)

    evh, evt, vvh, vvt, offh, offt, cnts = _classify(h, t)
    evh2 = evh.reshape(_NW * EV_NCH, CH)
    evt2 = evt.reshape(_NW * EV_NCH, CH)
    vvh2 = vvh.reshape(_NW * VV_NCH, CH)
    vvt2 = vvt.reshape(_NW * VV_NCH, CH)
    offh2 = offh.reshape(_NW * OFF_NCH, CH)
    offt2 = offt.reshape(_NW * OFF_NCH, CH)

    vpad = jnp.zeros((N_PAD - N_VISITS, DIM), jnp.float32)
    for _ in range(2):
        a2, bmax = _a2max_n(embp, w1t, w2t, b1r, b2r)
        M = jnp.max(bmax[:, 0, :], axis=0, keepdims=True)
        uw1 = _uw_n(a2, embp, M)
        acc1 = _seg_sum_ev(evh2, evt2, cnts, uw1).reshape(SROWS, 256)
        agg2 = _tc_fin1(acc1, time_p)
        a2b, bmax2 = _a2max_v(agg2, w1t, w2t, b1r, b2r)
        M2 = jnp.max(bmax2[:, 0, :], axis=0, keepdims=True)
        uw2 = _uw_v(a2b, agg2, M2)
        acc2 = _seg_sum_vv(vvh2, vvt2, cnts, uw2).reshape(SROWS, 256)
        agg = _tc_fin2(acc2)

        oacc = _offsets(offh2, offt2, cnts, offp)
        flat = oacc.reshape(_NW, ACC_R, DIM)[:, :BIN_W, :].reshape(N_PAD, DIM)
        offp = _tc_finoff(flat)
        embp = jnp.concatenate([agg[:N_VISITS], vpad], axis=0)
    return embp[:N_VISITS], offp[:N_VISITS]
